# Initial kernel scaffold; baseline (speedup 1.0000x reference)
import jax
import jax.numpy as jnp
from jax.experimental import pallas as pl


def kernel(video_embeds, audio_embeds, Wq, Wk):
    raise NotImplementedError("write your pallas kernel here")


# speedup vs baseline: 3.8659x; 3.8659x over previous
import functools

import jax
import jax.numpy as jnp
from jax.experimental import pallas as pl

_H = 12
_DH = 64
_KV = 256
_KA = 128


def _rank_ids_weights(x_row, L, K):
    f32 = jnp.float32
    x_col = x_row.reshape(L, 1)
    i_idx = jax.lax.broadcasted_iota(jnp.int32, (L, L), 0)
    j_idx = jax.lax.broadcasted_iota(jnp.int32, (L, L), 1)
    gt = x_row > x_col
    eq = x_row == x_col
    prec = jnp.logical_or(gt, jnp.logical_and(eq, j_idx < i_idx))
    rank = jnp.sum(prec.astype(f32), axis=1, keepdims=True)
    r_iota = jax.lax.broadcasted_iota(jnp.int32, (L, L), 1)
    onehot = rank.astype(jnp.int32) == r_iota
    i_col = jax.lax.broadcasted_iota(jnp.int32, (L, 1), 0).astype(f32)
    ids = jnp.sum(jnp.where(onehot, i_col, 0.0), axis=0, keepdims=True).astype(jnp.int32)
    w = jnp.where(rank < float(K), x_col, 0.0)
    return ids, w


def _stella_kernel(v_ref, a_ref, wq_ref, wk_ref, out_ref, idv_ref, ida_ref):
    f32 = jnp.float32
    v = v_ref[0]
    a = a_ref[0]
    wq = wq_ref[...]
    wk = wk_ref[...]
    L_V, D = v.shape
    L_A = a.shape[0]
    scale = _DH ** -0.5

    def mm(x, y, dims):
        return jax.lax.dot_general(x, y, (dims, ((), ())),
                                   preferred_element_type=f32)

    vq = mm(v, wq, ((1,), (0,)))
    vk = mm(v, wk, ((1,), (0,)))
    aq = mm(a, wq, ((1,), (0,)))
    ak = mm(a, wk, ((1,), (0,)))

    def rowsum(x):
        return jnp.sum(x, axis=1, keepdims=True)

    acc_av = jnp.zeros((8, L_V), f32)
    acc_va = jnp.zeros((8, L_A), f32)
    for h in range(_H):
        sl = slice(h * _DH, (h + 1) * _DH)
        aq_h, vk_h = aq[:, sl], vk[:, sl]
        vq_h, ak_h = vq[:, sl], ak[:, sl]

        log_av = mm(aq_h, vk_h, ((1,), (1,))) * scale
        m = jnp.max(log_av, axis=1, keepdims=True)
        e = jnp.exp(log_av - m)
        att = e / rowsum(e)
        acc_av = acc_av + jnp.sum(att.reshape(L_A // 8, 8, L_V), axis=0)

        log_va = mm(vq_h, ak_h, ((1,), (1,))) * scale
        m2 = jnp.max(log_va, axis=1, keepdims=True)
        e2 = jnp.exp(log_va - m2)
        att2 = e2 / rowsum(e2)
        acc_va = acc_va + jnp.sum(att2.reshape(L_V // 8, 8, L_A), axis=0)

    n_av = jnp.sum(acc_av, axis=0, keepdims=True)
    n_va = jnp.sum(acc_va, axis=0, keepdims=True)

    ids_v, w_v = _rank_ids_weights(n_av, L_V, _KV)
    ids_a, w_a = _rank_ids_weights(n_va, L_A, _KA)
    idv_ref[0] = ids_v
    ida_ref[0] = ids_a

    num_v = jax.lax.dot_general(w_v, vq, (((0,), (0,)), ((), ())),
                                preferred_element_type=f32)
    num_a = jax.lax.dot_general(w_a, aq, (((0,), (0,)), ((), ())),
                                preferred_element_type=f32)
    pooled_v = num_v / jnp.sum(w_v)
    pooled_a = num_a / jnp.sum(w_a)
    out_ref[0] = jnp.concatenate([pooled_v, pooled_a], axis=1)


@functools.partial(jax.jit, static_argnames=())
def kernel(video_embeds, audio_embeds, Wq, Wk):
    N, L_V, D = video_embeds.shape
    L_A = audio_embeds.shape[1]
    out_flat, idv, ida = pl.pallas_call(
        _stella_kernel,
        grid=(N,),
        in_specs=[
            pl.BlockSpec((1, L_V, D), lambda n: (n, 0, 0)),
            pl.BlockSpec((1, L_A, D), lambda n: (n, 0, 0)),
            pl.BlockSpec((D, D), lambda n: (0, 0)),
            pl.BlockSpec((D, D), lambda n: (0, 0)),
        ],
        out_specs=[
            pl.BlockSpec((1, 1, 2 * D), lambda n: (n, 0, 0)),
            pl.BlockSpec((1, 1, L_V), lambda n: (n, 0, 0)),
            pl.BlockSpec((1, 1, L_A), lambda n: (n, 0, 0)),
        ],
        out_shape=[
            jax.ShapeDtypeStruct((N, 1, 2 * D), jnp.float32),
            jax.ShapeDtypeStruct((N, 1, L_V), jnp.int32),
            jax.ShapeDtypeStruct((N, 1, L_A), jnp.int32),
        ],
    )(video_embeds, audio_embeds, Wq, Wk)
    out = out_flat.reshape(N, 2 * _H, _DH)
    return out, idv.reshape(N, L_V), ida.reshape(N, L_A)
